# Initial kernel scaffold; baseline (speedup 1.0000x reference)
import jax
import jax.numpy as jnp
from jax.experimental import pallas as pl


def kernel(x, edge_index, batch, W1, b1, W2, b2, Wl1, bl1, Wl2, bl2):
    raise NotImplementedError("write your pallas kernel here")


# speedup vs baseline: 6.9880x; 6.9880x over previous
import jax
import jax.numpy as jnp
from jax import lax
from jax.experimental import pallas as pl
from jax.experimental.pallas import tpu as pltpu
from jax.experimental.pallas import tpu_sc as plsc

N = 10000
E = 160000
D_IN = 386
H = 128
NG = 64

RB = 1000
NRB = N // RB

NSC = 2
NTILE = 16
NW = NSC * NTILE
CH = 128
CPW = 40
EP = NW * CPW * CH
NACC = 10112
ZCH = (128, 128, 128, 128, 120)
ORT = NACC // NTILE
RING = 2

_f32 = jnp.float32


def _sc_mesh():
    return plsc.VectorSubcoreMesh(core_axis_name="c", subcore_axis_name="s")


def _deg_body(dstr, orow, zrow, out, acc, dstv, obuf, zbuf, dsem):
    cid = lax.axis_index("c")
    sid = lax.axis_index("s")
    wid = cid * NTILE + sid
    pltpu.sync_copy(dstr.at[pl.ds(wid * CPW, CPW)], dstv)
    pltpu.sync_copy(zrow, zbuf)
    pltpu.sync_copy(orow, obuf)
    zb = sid * ORT
    off = 0
    for sz in ZCH:
        pltpu.sync_copy(zbuf.at[pl.ds(0, sz)], acc.at[pl.ds(zb + off, sz)])
        off += sz
    plsc.subcore_barrier()

    def fire(j, carry):
        pltpu.async_copy(obuf, acc.at[dstv.at[j]], dsem, add=True)
        return carry

    lax.fori_loop(0, CPW, fire, 0)

    def drain(j, carry):
        pltpu.make_async_copy(obuf, acc.at[dstv.at[j]], dsem).wait()
        return carry

    lax.fori_loop(0, CPW, drain, 0)
    plsc.subcore_barrier()
    ob = sid * ORT
    pltpu.sync_copy(acc.at[pl.ds(ob, ORT)], out.at[cid, pl.ds(ob, ORT)])


def _scat_body(g, srcr, dstr, zrow, out,
               acc, srcv, dstv, b0, b1,
               g0, g1, s0, s1):
    bufs = (b0, b1)
    gsem = (g0, g1)
    ssem = (s0, s1)
    cid = lax.axis_index("c")
    sid = lax.axis_index("s")
    wid = cid * NTILE + sid
    pltpu.sync_copy(srcr.at[pl.ds(wid * CPW, CPW)], srcv)
    pltpu.sync_copy(dstr.at[pl.ds(wid * CPW, CPW)], dstv)
    pltpu.sync_copy(zrow, b0)
    zb = sid * ORT
    off = 0
    for sz in ZCH:
        pltpu.sync_copy(b0.at[pl.ds(0, sz)], acc.at[pl.ds(zb + off, sz)])
        off += sz
    plsc.subcore_barrier()

    for b in range(RING):
        pltpu.async_copy(g.at[srcv.at[b]], bufs[b], gsem[b])

    def body(it, carry):
        j0 = it * RING
        for b in range(RING):
            pltpu.make_async_copy(g.at[srcv.at[j0 + b]], bufs[b], gsem[b]).wait()
            pltpu.async_copy(bufs[b], acc.at[dstv.at[j0 + b]], ssem[b], add=True)
        for b in range(RING):
            jn = j0 + RING + b
            pltpu.make_async_copy(bufs[b], acc.at[dstv.at[j0 + b]], ssem[b]).wait()

            @pl.when(jn < CPW)
            def _():
                pltpu.async_copy(g.at[srcv.at[jn]], bufs[b], gsem[b])

        return carry

    lax.fori_loop(0, CPW // RING, body, 0)
    plsc.subcore_barrier()
    ob = sid * ORT
    pltpu.sync_copy(acc.at[pl.ds(ob, ORT)], out.at[cid, pl.ds(ob, ORT)])


def _deg_of(d_ref):
    d = d_ref[0] + d_ref[1]
    return 1.0 + jnp.sum(d, axis=1, keepdims=True)


def _mm1_body(x_ref, w_ref, d_ref, h_ref, g_ref):
    h = jnp.dot(x_ref[...], w_ref[...], preferred_element_type=_f32)
    dinv = lax.rsqrt(_deg_of(d_ref))
    h_ref[...] = h
    g_ref[...] = h * dinv


def _mm2_body(a_ref, h1_ref, d_ref, w2_ref, b1_ref, h2_ref, g2_ref):
    deg = _deg_of(d_ref)
    dinv = lax.rsqrt(deg)
    z = dinv * (a_ref[0] + a_ref[1]) + h1_ref[...] / deg + b1_ref[...]
    z = jnp.maximum(z, 0.0)
    h2 = jnp.dot(z, w2_ref[...], preferred_element_type=_f32)
    h2_ref[...] = h2
    g2_ref[...] = h2 * dinv


def _final_body(a_ref, h2_ref, d_ref, b2_ref, bt_ref,
                wl1_ref, bl1_ref, wl2_ref, bl2_ref, out_ref, sums, cnts):
    i = pl.program_id(0)

    @pl.when(i == 0)
    def _():
        sums[...] = jnp.zeros_like(sums)
        cnts[...] = jnp.zeros_like(cnts)

    deg = _deg_of(d_ref)
    dinv = lax.rsqrt(deg)
    z = dinv * (a_ref[0] + a_ref[1]) + h2_ref[...] / deg + b2_ref[...]
    z = jnp.maximum(z, 0.0)
    bt = bt_ref[0, 0, :]
    gid = lax.broadcasted_iota(jnp.int32, (NG, RB), 0)
    oh = (gid == bt[None, :]).astype(_f32)
    sums[...] += jnp.dot(oh, z, preferred_element_type=_f32)
    cnts[...] += jnp.broadcast_to(jnp.sum(oh, axis=1, keepdims=True), (NG, H))

    @pl.when(i == NRB - 1)
    def _():
        pooled = sums[...] / jnp.maximum(cnts[...], 1.0)
        t = jnp.dot(pooled, wl1_ref[...], preferred_element_type=_f32)
        t = jnp.maximum(t + bl1_ref[...], 0.0)
        u = jnp.dot(t, wl2_ref[...], preferred_element_type=_f32) + bl2_ref[...]
        out_ref[...] = 1.0 / (1.0 + jnp.exp(-u))


def kernel(x, edge_index, batch, W1, b1, W2, b2, Wl1, bl1, Wl2, bl2):
    src = edge_index[0]
    dst = edge_index[1]
    pad = EP - E
    srcr = jnp.concatenate([src, jnp.zeros((pad,), jnp.int32)]).reshape(EP // CH, CH)
    dstr = jnp.concatenate([dst, jnp.full((pad,), N, jnp.int32)]).reshape(EP // CH, CH)
    zrow = jnp.zeros((CH, H), _f32)
    zrow8 = jnp.zeros((CH, 8), _f32)
    orow8 = jnp.concatenate([jnp.ones((CH, 1), _f32), jnp.zeros((CH, 7), _f32)], axis=1)

    deg_call = pl.kernel(
        _deg_body,
        out_type=jax.ShapeDtypeStruct((NSC, NACC, 8), _f32),
        mesh=_sc_mesh(),
        scratch_types=[
            pltpu.VMEM_SHARED((NACC, 8), _f32),
            pltpu.VMEM((CPW, CH), jnp.int32),
            pltpu.VMEM((CH, 8), _f32),
            pltpu.VMEM((CH, 8), _f32),
            pltpu.SemaphoreType.DMA,
        ],
    )
    deg8 = deg_call(dstr, orow8, zrow8)

    row_spec = pl.BlockSpec((RB, H), lambda i: (i, 0))
    d8_spec = pl.BlockSpec((NSC, RB, 8), lambda i: (0, i, 0))
    acc_spec = pl.BlockSpec((NSC, RB, H), lambda i: (0, i, 0))
    full = lambda shape: pl.BlockSpec(shape, lambda i: tuple(0 for _ in shape))

    mm1 = pl.pallas_call(
        _mm1_body,
        grid=(NRB,),
        in_specs=[
            pl.BlockSpec((RB, D_IN), lambda i: (i, 0)),
            full((D_IN, H)),
            d8_spec,
        ],
        out_specs=[row_spec, row_spec],
        out_shape=[
            jax.ShapeDtypeStruct((N, H), _f32),
            jax.ShapeDtypeStruct((N, H), _f32),
        ],
    )
    h1, g1 = mm1(x, W1, deg8)

    scat_call = pl.kernel(
        _scat_body,
        out_type=jax.ShapeDtypeStruct((NSC, NACC, H), _f32),
        mesh=_sc_mesh(),
        scratch_types=[
            pltpu.VMEM_SHARED((NACC, H), _f32),
            pltpu.VMEM((CPW, CH), jnp.int32),
            pltpu.VMEM((CPW, CH), jnp.int32),
        ] + [pltpu.VMEM((CH, H), _f32)] * RING
          + [pltpu.SemaphoreType.DMA] * (2 * RING),
    )
    acc1 = scat_call(g1, srcr, dstr, zrow)

    mm2 = pl.pallas_call(
        _mm2_body,
        grid=(NRB,),
        in_specs=[
            acc_spec, row_spec, d8_spec,
            full((H, H)), full((1, H)),
        ],
        out_specs=[row_spec, row_spec],
        out_shape=[
            jax.ShapeDtypeStruct((N, H), _f32),
            jax.ShapeDtypeStruct((N, H), _f32),
        ],
    )
    h2, g2 = mm2(acc1, h1, deg8, W2, b1.reshape(1, H))

    acc2 = scat_call(g2, srcr, dstr, zrow)

    wl1p = jnp.pad(Wl1, ((0, 0), (0, H - Wl1.shape[1])))
    bl1p = jnp.pad(bl1, (0, H - bl1.shape[0])).reshape(1, H)
    wl2p = jnp.pad(Wl2, ((0, H - Wl2.shape[0]), (0, H - Wl2.shape[1])))
    bl2p = jnp.pad(bl2, (0, H - bl2.shape[0])).reshape(1, H)
    batch3 = batch.reshape(NRB, 1, RB)

    final = pl.pallas_call(
        _final_body,
        grid=(NRB,),
        in_specs=[
            acc_spec, row_spec, d8_spec,
            full((1, H)),
            pl.BlockSpec((1, 1, RB), lambda i: (i, 0, 0)),
            full((H, H)), full((1, H)), full((H, H)), full((1, H)),
        ],
        out_specs=pl.BlockSpec((NG, H), lambda i: (0, 0)),
        out_shape=jax.ShapeDtypeStruct((NG, H), _f32),
        scratch_shapes=[pltpu.VMEM((NG, H), _f32), pltpu.VMEM((NG, H), _f32)],
    )
    outp = final(acc2, h2, deg8, b2.reshape(1, H), batch3,
                 wl1p, bl1p, wl2p, bl2p)
    return outp[:, :5]


# speedup vs baseline: 8.0805x; 1.1563x over previous
import jax
import jax.numpy as jnp
from jax import lax
from jax.experimental import pallas as pl
from jax.experimental.pallas import tpu as pltpu
from jax.experimental.pallas import tpu_sc as plsc

N = 10000
E = 160000
D_IN = 386
H = 128
NG = 64

RB = 1000
NRB = N // RB

NSC = 2
NTILE = 16
NW = NSC * NTILE
CH = 128
CPW = 40
EP = NW * CPW * CH
NACC = 10112
ZCH = (128, 128, 128, 128, 120)
ORT = NACC // NTILE
RING = 2
DSEM = 4

_f32 = jnp.float32


def _sc_mesh():
    return plsc.VectorSubcoreMesh(core_axis_name="c", subcore_axis_name="s")


def _deg_body(dstr, ones_in, zrow, out, acc, dstv, obuf, zbuf, *sems):
    cid = lax.axis_index("c")
    sid = lax.axis_index("s")
    wid = cid * NTILE + sid
    pltpu.sync_copy(dstr.at[pl.ds(wid * CPW, CPW)], dstv)
    pltpu.sync_copy(ones_in, obuf)
    pltpu.sync_copy(zrow, zbuf)
    zb = sid * ORT
    off = 0
    for sz in ZCH:
        pltpu.sync_copy(zbuf.at[pl.ds(0, sz)], acc.at[pl.ds(zb + off, sz)])
        off += sz
    plsc.subcore_barrier()

    for j in range(CPW):
        if j >= DSEM:
            pltpu.make_async_copy(
                obuf, acc.at[dstv.at[j - DSEM]], sems[j % DSEM]
            ).wait()
        pltpu.async_copy(obuf, acc.at[dstv.at[j]], sems[j % DSEM], add=True)
    for j in range(CPW - DSEM, CPW):
        pltpu.make_async_copy(obuf, acc.at[dstv.at[j]], sems[j % DSEM]).wait()

    plsc.subcore_barrier()
    ob = sid * ORT
    pltpu.sync_copy(acc.at[pl.ds(ob, ORT)], out.at[cid, pl.ds(ob, ORT)])


def _scat_body(g, srcr, dstr, zrow, out,
               acc, srcv, dstv, b0, b1,
               g0, g1, s0, s1):
    bufs = (b0, b1)
    gsem = (g0, g1)
    ssem = (s0, s1)
    cid = lax.axis_index("c")
    sid = lax.axis_index("s")
    wid = cid * NTILE + sid
    pltpu.sync_copy(srcr.at[pl.ds(wid * CPW, CPW)], srcv)
    pltpu.sync_copy(dstr.at[pl.ds(wid * CPW, CPW)], dstv)
    pltpu.sync_copy(zrow, b0)
    zb = sid * ORT
    off = 0
    for sz in ZCH:
        pltpu.sync_copy(b0.at[pl.ds(0, sz)], acc.at[pl.ds(zb + off, sz)])
        off += sz
    plsc.subcore_barrier()

    for b in range(RING):
        pltpu.async_copy(g.at[srcv.at[b]], bufs[b], gsem[b])

    def body(it, carry):
        j0 = it * RING
        for b in range(RING):
            pltpu.make_async_copy(g.at[srcv.at[j0 + b]], bufs[b], gsem[b]).wait()
            pltpu.async_copy(bufs[b], acc.at[dstv.at[j0 + b]], ssem[b], add=True)
        for b in range(RING):
            jn = j0 + RING + b
            pltpu.make_async_copy(bufs[b], acc.at[dstv.at[j0 + b]], ssem[b]).wait()

            @pl.when(jn < CPW)
            def _():
                pltpu.async_copy(g.at[srcv.at[jn]], bufs[b], gsem[b])

        return carry

    lax.fori_loop(0, CPW // RING, body, 0)
    plsc.subcore_barrier()
    ob = sid * ORT
    pltpu.sync_copy(acc.at[pl.ds(ob, ORT)], out.at[cid, pl.ds(ob, ORT)])


def _deg_of(d_ref):
    return 1.0 + d_ref[0, :, 0:1] + d_ref[1, :, 0:1]


def _mm1_body(x_ref, w_ref, d_ref, h_ref, g_ref):
    h = jnp.dot(x_ref[...], w_ref[...], preferred_element_type=_f32)
    dinv = lax.rsqrt(_deg_of(d_ref))
    h_ref[...] = h
    g_ref[...] = h * dinv


def _mm2_body(a_ref, h1_ref, d_ref, w2_ref, b1_ref, h2_ref, g2_ref):
    deg = _deg_of(d_ref)
    dinv = lax.rsqrt(deg)
    z = dinv * (a_ref[0] + a_ref[1]) + h1_ref[...] / deg + b1_ref[...]
    z = jnp.maximum(z, 0.0)
    h2 = jnp.dot(z, w2_ref[...], preferred_element_type=_f32)
    h2_ref[...] = h2
    g2_ref[...] = h2 * dinv


def _final_body(a_ref, h2_ref, d_ref, b2_ref, bt_ref,
                wl1_ref, bl1_ref, wl2_ref, bl2_ref, out_ref, sums, cnts):
    i = pl.program_id(0)

    @pl.when(i == 0)
    def _():
        sums[...] = jnp.zeros_like(sums)
        cnts[...] = jnp.zeros_like(cnts)

    deg = _deg_of(d_ref)
    dinv = lax.rsqrt(deg)
    z = dinv * (a_ref[0] + a_ref[1]) + h2_ref[...] / deg + b2_ref[...]
    z = jnp.maximum(z, 0.0)
    bt = bt_ref[0, 0, :]
    gid = lax.broadcasted_iota(jnp.int32, (NG, RB), 0)
    oh = (gid == bt[None, :]).astype(_f32)
    sums[...] += jnp.dot(oh, z, preferred_element_type=_f32)
    cnts[...] += jnp.broadcast_to(jnp.sum(oh, axis=1, keepdims=True), (NG, H))

    @pl.when(i == NRB - 1)
    def _():
        pooled = sums[...] / jnp.maximum(cnts[...], 1.0)
        t = jnp.dot(pooled, wl1_ref[...], preferred_element_type=_f32)
        t = jnp.maximum(t + bl1_ref[...], 0.0)
        u = jnp.dot(t, wl2_ref[...], preferred_element_type=_f32) + bl2_ref[...]
        out_ref[...] = 1.0 / (1.0 + jnp.exp(-u))


def kernel(x, edge_index, batch, W1, b1, W2, b2, Wl1, bl1, Wl2, bl2):
    src = edge_index[0]
    dst = edge_index[1]
    pad = EP - E
    srcr = jnp.concatenate([src, jnp.zeros((pad,), jnp.int32)]).reshape(EP // CH, CH)
    dstr = jnp.concatenate([dst, jnp.full((pad,), N, jnp.int32)]).reshape(EP // CH, CH)
    zrow = jnp.zeros((CH, H), _f32)

    ones_in = jnp.ones((CH, H), _f32)

    deg_call = pl.kernel(
        _deg_body,
        out_type=jax.ShapeDtypeStruct((NSC, NACC, H), _f32),
        mesh=_sc_mesh(),
        scratch_types=[
            pltpu.VMEM_SHARED((NACC, H), _f32),
            pltpu.VMEM((CPW, CH), jnp.int32),
            pltpu.VMEM((CH, H), _f32),
            pltpu.VMEM((CH, H), _f32),
        ] + [pltpu.SemaphoreType.DMA] * DSEM,
    )
    deg8 = deg_call(dstr, ones_in, zrow)

    row_spec = pl.BlockSpec((RB, H), lambda i: (i, 0))
    d8_spec = pl.BlockSpec((NSC, RB, H), lambda i: (0, i, 0))
    acc_spec = pl.BlockSpec((NSC, RB, H), lambda i: (0, i, 0))
    full = lambda shape: pl.BlockSpec(shape, lambda i: tuple(0 for _ in shape))

    mm1 = pl.pallas_call(
        _mm1_body,
        grid=(NRB,),
        in_specs=[
            pl.BlockSpec((RB, D_IN), lambda i: (i, 0)),
            full((D_IN, H)),
            d8_spec,
        ],
        out_specs=[row_spec, row_spec],
        out_shape=[
            jax.ShapeDtypeStruct((N, H), _f32),
            jax.ShapeDtypeStruct((N, H), _f32),
        ],
    )
    h1, g1 = mm1(x, W1, deg8)

    scat_call = pl.kernel(
        _scat_body,
        out_type=jax.ShapeDtypeStruct((NSC, NACC, H), _f32),
        mesh=_sc_mesh(),
        scratch_types=[
            pltpu.VMEM_SHARED((NACC, H), _f32),
            pltpu.VMEM((CPW, CH), jnp.int32),
            pltpu.VMEM((CPW, CH), jnp.int32),
        ] + [pltpu.VMEM((CH, H), _f32)] * RING
          + [pltpu.SemaphoreType.DMA] * (2 * RING),
    )
    acc1 = scat_call(g1, srcr, dstr, zrow)

    mm2 = pl.pallas_call(
        _mm2_body,
        grid=(NRB,),
        in_specs=[
            acc_spec, row_spec, d8_spec,
            full((H, H)), full((1, H)),
        ],
        out_specs=[row_spec, row_spec],
        out_shape=[
            jax.ShapeDtypeStruct((N, H), _f32),
            jax.ShapeDtypeStruct((N, H), _f32),
        ],
    )
    h2, g2 = mm2(acc1, h1, deg8, W2, b1.reshape(1, H))

    acc2 = scat_call(g2, srcr, dstr, zrow)

    wl1p = jnp.pad(Wl1, ((0, 0), (0, H - Wl1.shape[1])))
    bl1p = jnp.pad(bl1, (0, H - bl1.shape[0])).reshape(1, H)
    wl2p = jnp.pad(Wl2, ((0, H - Wl2.shape[0]), (0, H - Wl2.shape[1])))
    bl2p = jnp.pad(bl2, (0, H - bl2.shape[0])).reshape(1, H)
    batch3 = batch.reshape(NRB, 1, RB)

    final = pl.pallas_call(
        _final_body,
        grid=(NRB,),
        in_specs=[
            acc_spec, row_spec, d8_spec,
            full((1, H)),
            pl.BlockSpec((1, 1, RB), lambda i: (i, 0, 0)),
            full((H, H)), full((1, H)), full((H, H)), full((1, H)),
        ],
        out_specs=pl.BlockSpec((NG, H), lambda i: (0, 0)),
        out_shape=jax.ShapeDtypeStruct((NG, H), _f32),
        scratch_shapes=[pltpu.VMEM((NG, H), _f32), pltpu.VMEM((NG, H), _f32)],
    )
    outp = final(acc2, h2, deg8, b2.reshape(1, H), batch3,
                 wl1p, bl1p, wl2p, bl2p)
    return outp[:, :5]
